# baseline (speedup 1.0000x reference)
import functools

import jax
import jax.numpy as jnp
from jax import lax
from jax.experimental import pallas as pl
from jax.experimental.pallas import tpu as pltpu
from jax.experimental.pallas import tpu_sc as plsc

N = 10000
E = 320000
D = 128

NPAD = 10240
NW = 32
STRIPE = NPAD // 16

C = 64
CH = 160
EPAD = NW * CH * C

CD = 128
CHD = EPAD // (NW * CD)

_mesh = plsc.VectorSubcoreMesh(core_axis_name="c", subcore_axis_name="s")


@functools.partial(
    pl.kernel,
    out_type=jax.ShapeDtypeStruct((2, NPAD), jnp.float32),
    mesh=_mesh,
    scratch_types=[
        pltpu.VMEM((CHD, CD), jnp.int32),
        pltpu.VMEM((CD,), jnp.float32),
        pltpu.VMEM_SHARED((NPAD,), jnp.float32),
        pltpu.SemaphoreType.DMA,
    ],
)
def _deg_kernel(dst_hbm, ones_hbm, zn_hbm, out_hbm, dst_all, ones_v, deg_sh, sem):
    c = lax.axis_index("c")
    s = lax.axis_index("s")
    wid = s * 2 + c
    pltpu.sync_copy(zn_hbm.at[pl.ds(s * STRIPE, STRIPE)],
                    deg_sh.at[pl.ds(s * STRIPE, STRIPE)])
    pltpu.sync_copy(dst_hbm.at[wid], dst_all)
    pltpu.sync_copy(ones_hbm, ones_v)
    plsc.subcore_barrier()

    def fire(j, carry):
        pltpu.async_copy(ones_v, deg_sh.at[dst_all.at[j]], sem, add=True)
        return carry

    def drain(j, carry):
        pltpu.make_async_copy(ones_v, deg_sh.at[pl.ds(0, CD)], sem).wait()
        return carry

    lax.fori_loop(0, CHD, fire, 0)
    lax.fori_loop(0, CHD, drain, 0)
    plsc.subcore_barrier()
    pltpu.sync_copy(deg_sh.at[pl.ds(s * STRIPE, STRIPE)],
                    out_hbm.at[c, pl.ds(s * STRIPE, STRIPE)])


@functools.partial(
    pl.kernel,
    out_type=jax.ShapeDtypeStruct((2, NPAD, D), jnp.float32),
    mesh=_mesh,
    scratch_types=[
        pltpu.VMEM((CH * C,), jnp.int32),
        pltpu.VMEM((CH, C), jnp.int32),
        pltpu.VMEM((2, C, D), jnp.float32),
        pltpu.VMEM_SHARED((NPAD, D), jnp.float32),
        pltpu.SemaphoreType.DMA,
        pltpu.SemaphoreType.DMA,
    ],
)
def _edge_kernel(g_hbm, src_hbm, dst_hbm, znd_hbm, out_hbm,
                 src_all, dst_all, rows, accum_sh, gsem, ssem):
    c = lax.axis_index("c")
    s = lax.axis_index("s")
    wid = s * 2 + c
    pltpu.sync_copy(znd_hbm.at[pl.ds(s * STRIPE, STRIPE)],
                    accum_sh.at[pl.ds(s * STRIPE, STRIPE)])
    pltpu.sync_copy(src_hbm.at[wid], src_all)
    pltpu.sync_copy(dst_hbm.at[wid], dst_all)
    plsc.subcore_barrier()

    def g_start(j, b):
        pltpu.async_copy(g_hbm.at[src_all.at[pl.ds(j * C, C)]],
                         rows.at[b], gsem)

    def g_wait(b):
        pltpu.make_async_copy(g_hbm.at[pl.ds(0, C)], rows.at[b], gsem).wait()

    def s_start(j, b):
        pltpu.async_copy(rows.at[b], accum_sh.at[dst_all.at[j]],
                         ssem, add=True)

    def s_wait(b):
        pltpu.make_async_copy(rows.at[b], accum_sh.at[pl.ds(0, C)],
                              ssem).wait()

    g_start(0, 0)
    g_wait(0)
    g_start(1, 1)
    s_start(0, 0)
    g_wait(1)
    s_wait(0)
    g_start(2, 0)
    s_start(1, 1)

    def steady(t, carry):
        j = 2 * t
        g_wait(0)
        s_wait(1)
        g_start(j + 1, 1)
        s_start(j, 0)
        g_wait(1)
        s_wait(0)
        g_start(j + 2, 0)
        s_start(j + 1, 1)
        return carry

    lax.fori_loop(1, CH // 2 - 1, steady, 0)

    g_wait(0)
    s_wait(1)
    g_start(CH - 1, 1)
    s_start(CH - 2, 0)
    g_wait(1)
    s_wait(0)
    s_start(CH - 1, 1)
    s_wait(1)

    plsc.subcore_barrier()
    pltpu.sync_copy(accum_sh.at[pl.ds(s * STRIPE, STRIPE)],
                    out_hbm.at[c, pl.ds(s * STRIPE, STRIPE)])


BR = 512


def _k1_body(degT_ref, x_ref, w_ref, dinv_ref, g_ref):
    deg = degT_ref[:, 0:1] + degT_ref[:, 1:2] + 1.0
    dinv = lax.rsqrt(deg)
    dinv_ref[...] = dinv
    h = jnp.dot(x_ref[...], w_ref[...], preferred_element_type=jnp.float32)
    g_ref[...] = h * dinv


def _k1(degT, x_pad, w):
    return pl.pallas_call(
        _k1_body,
        grid=(NPAD // BR,),
        in_specs=[
            pl.BlockSpec((BR, 2), lambda i: (i, 0)),
            pl.BlockSpec((BR, D), lambda i: (i, 0)),
            pl.BlockSpec((D, D), lambda i: (0, 0)),
        ],
        out_specs=[
            pl.BlockSpec((BR, 1), lambda i: (i, 0)),
            pl.BlockSpec((BR, D), lambda i: (i, 0)),
        ],
        out_shape=[
            jax.ShapeDtypeStruct((NPAD, 1), jnp.float32),
            jax.ShapeDtypeStruct((NPAD, D), jnp.float32),
        ],
    )(degT, x_pad, w)


def _k2_body(p_ref, g_ref, dinv_ref, b_ref, w_ref, gout_ref):
    a = p_ref[0] + p_ref[1] + g_ref[...]
    y = jnp.maximum(a * dinv_ref[...] + b_ref[...], 0.0)
    gout_ref[...] = jnp.dot(y, w_ref[...],
                            preferred_element_type=jnp.float32) * dinv_ref[...]


def _k2(p, g, dinv, b, w):
    return pl.pallas_call(
        _k2_body,
        grid=(NPAD // BR,),
        in_specs=[
            pl.BlockSpec((2, BR, D), lambda i: (0, i, 0)),
            pl.BlockSpec((BR, D), lambda i: (i, 0)),
            pl.BlockSpec((BR, 1), lambda i: (i, 0)),
            pl.BlockSpec((1, D), lambda i: (0, 0)),
            pl.BlockSpec((D, D), lambda i: (0, 0)),
        ],
        out_specs=pl.BlockSpec((BR, D), lambda i: (i, 0)),
        out_shape=jax.ShapeDtypeStruct((NPAD, D), jnp.float32),
    )(p, g, dinv, b, w)


def _k3_body(p_ref, g_ref, dinv_ref, b_ref, y_ref):
    a = p_ref[0] + p_ref[1] + g_ref[...]
    y_ref[...] = jnp.maximum(a * dinv_ref[...] + b_ref[...], 0.0)


def _k3(p, g, dinv, b):
    return pl.pallas_call(
        _k3_body,
        grid=(NPAD // BR,),
        in_specs=[
            pl.BlockSpec((2, BR, D), lambda i: (0, i, 0)),
            pl.BlockSpec((BR, D), lambda i: (i, 0)),
            pl.BlockSpec((BR, 1), lambda i: (i, 0)),
            pl.BlockSpec((1, D), lambda i: (0, 0)),
        ],
        out_specs=pl.BlockSpec((BR, D), lambda i: (i, 0)),
        out_shape=jax.ShapeDtypeStruct((NPAD, D), jnp.float32),
    )(p, g, dinv, b)


def kernel(x, edge_index, W1, b1, W2, b2):
    src = edge_index[0]
    dst = edge_index[1]
    pad = EPAD - E
    fill = jnp.full((pad,), N, jnp.int32)
    srcp = jnp.concatenate([src, fill]).reshape(NW, CH * C)
    dstp = jnp.concatenate([dst, fill]).reshape(NW, CH, C)
    dstd = jnp.concatenate([dst, fill]).reshape(NW, CHD, CD)
    x_pad = jnp.pad(x, ((0, NPAD - N), (0, 0)))
    zeros_nd = jnp.zeros((NPAD, D), jnp.float32)
    zeros_n = jnp.zeros((NPAD,), jnp.float32)
    ones_c = jnp.ones((CD,), jnp.float32)

    degp = _deg_kernel(dstd, ones_c, zeros_n)
    degT = degp.T
    b1r = b1.reshape(1, D)
    b2r = b2.reshape(1, D)

    dinv, g1 = _k1(degT, x_pad, W1)
    p1 = _edge_kernel(g1, srcp, dstp, zeros_nd)
    g2 = _k2(p1, g1, dinv, b1r, W2)
    p2 = _edge_kernel(g2, srcp, dstp, zeros_nd)
    y = _k3(p2, g2, dinv, b2r)
    return y[:N]


# speedup vs baseline: 1.0702x; 1.0702x over previous
import functools

import jax
import jax.numpy as jnp
from jax import lax
from jax.experimental import pallas as pl
from jax.experimental.pallas import tpu as pltpu
from jax.experimental.pallas import tpu_sc as plsc

N = 10000
E = 320000
D = 128

NPAD = 10240
NW = 32
STRIPE = NPAD // 16

C = 128
CH = 80
EPAD = NW * CH * C

CD = 128
CHD = EPAD // (NW * CD)

_mesh = plsc.VectorSubcoreMesh(core_axis_name="c", subcore_axis_name="s")


@functools.partial(
    pl.kernel,
    out_type=jax.ShapeDtypeStruct((2, NPAD), jnp.float32),
    mesh=_mesh,
    scratch_types=[
        pltpu.VMEM((CHD, CD), jnp.int32),
        pltpu.VMEM((CD,), jnp.float32),
        pltpu.VMEM_SHARED((NPAD,), jnp.float32),
        pltpu.SemaphoreType.DMA,
    ],
)
def _deg_kernel(dst_hbm, ones_hbm, zn_hbm, out_hbm, dst_all, ones_v, deg_sh, sem):
    c = lax.axis_index("c")
    s = lax.axis_index("s")
    wid = s * 2 + c
    pltpu.sync_copy(zn_hbm.at[pl.ds(s * STRIPE, STRIPE)],
                    deg_sh.at[pl.ds(s * STRIPE, STRIPE)])
    pltpu.sync_copy(dst_hbm.at[wid], dst_all)
    pltpu.sync_copy(ones_hbm, ones_v)
    plsc.subcore_barrier()

    def fire(j, carry):
        pltpu.async_copy(ones_v, deg_sh.at[dst_all.at[j]], sem, add=True)
        return carry

    def drain(j, carry):
        pltpu.make_async_copy(ones_v, deg_sh.at[pl.ds(0, CD)], sem).wait()
        return carry

    lax.fori_loop(0, CHD, fire, 0)
    lax.fori_loop(0, CHD, drain, 0)
    plsc.subcore_barrier()
    pltpu.sync_copy(deg_sh.at[pl.ds(s * STRIPE, STRIPE)],
                    out_hbm.at[c, pl.ds(s * STRIPE, STRIPE)])


@functools.partial(
    pl.kernel,
    out_type=jax.ShapeDtypeStruct((2, NPAD, D), jnp.float32),
    mesh=_mesh,
    scratch_types=[
        pltpu.VMEM((2, C), jnp.int32),
        pltpu.VMEM((CH, C), jnp.int32),
        pltpu.VMEM((2, C, D), jnp.float32),
        pltpu.VMEM_SHARED((NPAD, D), jnp.float32),
        pltpu.SemaphoreType.DMA,
        pltpu.SemaphoreType.DMA,
        pltpu.SemaphoreType.DMA,
        pltpu.SemaphoreType.DMA,
    ],
)
def _edge_kernel(g_hbm, src_hbm, dst_hbm, znd_hbm, out_hbm,
                 src_i, dst_all, rows, accum_sh, gsem, ssem, isem0, isem1):
    c = lax.axis_index("c")
    s = lax.axis_index("s")
    wid = s * 2 + c
    pltpu.sync_copy(znd_hbm.at[pl.ds(s * STRIPE, STRIPE)],
                    accum_sh.at[pl.ds(s * STRIPE, STRIPE)])
    pltpu.sync_copy(dst_hbm.at[wid], dst_all)
    plsc.subcore_barrier()

    isems = (isem0, isem1)

    def i_start(j, b):
        pltpu.async_copy(src_hbm.at[wid, pl.ds(j * C, C)], src_i.at[b],
                         isems[b])

    def i_wait(b):
        pltpu.make_async_copy(src_hbm.at[0, pl.ds(0, C)],
                              src_i.at[b], isems[b]).wait()

    def g_start(b):
        pltpu.async_copy(g_hbm.at[src_i.at[b]], rows.at[b], gsem)

    def g_wait(b):
        pltpu.make_async_copy(g_hbm.at[pl.ds(0, C)], rows.at[b], gsem).wait()

    def s_start(j, b):
        pltpu.async_copy(rows.at[b], accum_sh.at[dst_all.at[j]],
                         ssem, add=True)

    def s_wait(b):
        pltpu.make_async_copy(rows.at[b], accum_sh.at[pl.ds(0, C)],
                              ssem).wait()

    i_start(0, 0)
    i_wait(0)
    g_start(0)
    i_start(1, 1)
    g_wait(0)
    i_start(2, 0)
    i_wait(1)
    g_start(1)
    s_start(0, 0)
    g_wait(1)
    i_start(3, 1)
    s_wait(0)
    i_wait(0)
    g_start(0)
    s_start(1, 1)

    def steady(t, carry):
        j = 2 * t
        g_wait(0)
        i_start(j + 2, 0)
        s_wait(1)
        i_wait(1)
        g_start(1)
        s_start(j, 0)
        g_wait(1)
        i_start(j + 3, 1)
        s_wait(0)
        i_wait(0)
        g_start(0)
        s_start(j + 1, 1)
        return carry

    lax.fori_loop(1, CH // 2 - 1, steady, 0)

    g_wait(0)
    s_wait(1)
    i_wait(1)
    g_start(1)
    s_start(CH - 2, 0)
    g_wait(1)
    s_wait(0)
    s_start(CH - 1, 1)
    s_wait(1)

    plsc.subcore_barrier()
    pltpu.sync_copy(accum_sh.at[pl.ds(s * STRIPE, STRIPE)],
                    out_hbm.at[c, pl.ds(s * STRIPE, STRIPE)])


BR = 512


def _k1_body(degT_ref, x_ref, w_ref, dinv_ref, g_ref):
    deg = degT_ref[:, 0:1] + degT_ref[:, 1:2] + 1.0
    dinv = lax.rsqrt(deg)
    dinv_ref[...] = dinv
    h = jnp.dot(x_ref[...], w_ref[...], preferred_element_type=jnp.float32)
    g_ref[...] = h * dinv


def _k1(degT, x_pad, w):
    return pl.pallas_call(
        _k1_body,
        grid=(NPAD // BR,),
        in_specs=[
            pl.BlockSpec((BR, 2), lambda i: (i, 0)),
            pl.BlockSpec((BR, D), lambda i: (i, 0)),
            pl.BlockSpec((D, D), lambda i: (0, 0)),
        ],
        out_specs=[
            pl.BlockSpec((BR, 1), lambda i: (i, 0)),
            pl.BlockSpec((BR, D), lambda i: (i, 0)),
        ],
        out_shape=[
            jax.ShapeDtypeStruct((NPAD, 1), jnp.float32),
            jax.ShapeDtypeStruct((NPAD, D), jnp.float32),
        ],
    )(degT, x_pad, w)


def _k2_body(p_ref, g_ref, dinv_ref, b_ref, w_ref, gout_ref):
    a = p_ref[0] + p_ref[1] + g_ref[...]
    y = jnp.maximum(a * dinv_ref[...] + b_ref[...], 0.0)
    gout_ref[...] = jnp.dot(y, w_ref[...],
                            preferred_element_type=jnp.float32) * dinv_ref[...]


def _k2(p, g, dinv, b, w):
    return pl.pallas_call(
        _k2_body,
        grid=(NPAD // BR,),
        in_specs=[
            pl.BlockSpec((2, BR, D), lambda i: (0, i, 0)),
            pl.BlockSpec((BR, D), lambda i: (i, 0)),
            pl.BlockSpec((BR, 1), lambda i: (i, 0)),
            pl.BlockSpec((1, D), lambda i: (0, 0)),
            pl.BlockSpec((D, D), lambda i: (0, 0)),
        ],
        out_specs=pl.BlockSpec((BR, D), lambda i: (i, 0)),
        out_shape=jax.ShapeDtypeStruct((NPAD, D), jnp.float32),
    )(p, g, dinv, b, w)


def _k3_body(p_ref, g_ref, dinv_ref, b_ref, y_ref):
    a = p_ref[0] + p_ref[1] + g_ref[...]
    y_ref[...] = jnp.maximum(a * dinv_ref[...] + b_ref[...], 0.0)


def _k3(p, g, dinv, b):
    return pl.pallas_call(
        _k3_body,
        grid=(NPAD // BR,),
        in_specs=[
            pl.BlockSpec((2, BR, D), lambda i: (0, i, 0)),
            pl.BlockSpec((BR, D), lambda i: (i, 0)),
            pl.BlockSpec((BR, 1), lambda i: (i, 0)),
            pl.BlockSpec((1, D), lambda i: (0, 0)),
        ],
        out_specs=pl.BlockSpec((BR, D), lambda i: (i, 0)),
        out_shape=jax.ShapeDtypeStruct((NPAD, D), jnp.float32),
    )(p, g, dinv, b)


def kernel(x, edge_index, W1, b1, W2, b2):
    src = edge_index[0]
    dst = edge_index[1]
    pad = EPAD - E
    fill = jnp.full((pad,), N, jnp.int32)
    srcp = jnp.concatenate([src, fill]).reshape(NW, CH * C)
    dstp = jnp.concatenate([dst, fill]).reshape(NW, CH, C)
    dstd = jnp.concatenate([dst, fill]).reshape(NW, CHD, CD)
    x_pad = jnp.pad(x, ((0, NPAD - N), (0, 0)))
    zeros_nd = jnp.zeros((NPAD, D), jnp.float32)
    zeros_n = jnp.zeros((NPAD,), jnp.float32)
    ones_c = jnp.ones((CD,), jnp.float32)

    degp = _deg_kernel(dstd, ones_c, zeros_n)
    degT = degp.T
    b1r = b1.reshape(1, D)
    b2r = b2.reshape(1, D)

    dinv, g1 = _k1(degT, x_pad, W1)
    p1 = _edge_kernel(g1, srcp, dstp, zeros_nd)
    g2 = _k2(p1, g1, dinv, b1r, W2)
    p2 = _edge_kernel(g2, srcp, dstp, zeros_nd)
    y = _k3(p2, g2, dinv, b2r)
    return y[:N]


# speedup vs baseline: 3.4376x; 3.2120x over previous
import functools

import jax
import jax.numpy as jnp
from jax import lax
from jax.experimental import pallas as pl
from jax.experimental.pallas import tpu as pltpu
from jax.experimental.pallas import tpu_sc as plsc

N = 10000
E = 320000
D = 128

NPAD = 10240
NW = 32
STRIPE = NPAD // 16

C = 128
CH = 80
EPAD = NW * CH * C

CD = 128
CHD = EPAD // (NW * CD)

_mesh = plsc.VectorSubcoreMesh(core_axis_name="c", subcore_axis_name="s")


@functools.partial(
    pl.kernel,
    out_type=jax.ShapeDtypeStruct((2, NPAD), jnp.float32),
    mesh=_mesh,
    scratch_types=[
        pltpu.VMEM((CHD, CD), jnp.int32),
        pltpu.VMEM((CD,), jnp.float32),
        pltpu.VMEM_SHARED((NPAD,), jnp.float32),
        pltpu.SemaphoreType.DMA,
    ],
)
def _deg_kernel(dst_hbm, ones_hbm, zn_hbm, out_hbm, dst_all, ones_v, deg_sh, sem):
    c = lax.axis_index("c")
    s = lax.axis_index("s")
    wid = s * 2 + c
    pltpu.sync_copy(zn_hbm.at[pl.ds(s * STRIPE, STRIPE)],
                    deg_sh.at[pl.ds(s * STRIPE, STRIPE)])
    pltpu.sync_copy(dst_hbm.at[wid], dst_all)
    pltpu.sync_copy(ones_hbm, ones_v)
    plsc.subcore_barrier()

    def fire(j, carry):
        pltpu.async_copy(ones_v, deg_sh.at[dst_all.at[j]], sem, add=True)
        return carry

    def drain(j, carry):
        pltpu.make_async_copy(ones_v, deg_sh.at[pl.ds(0, CD)], sem).wait()
        return carry

    lax.fori_loop(0, CHD, fire, 0)
    lax.fori_loop(0, CHD, drain, 0)
    plsc.subcore_barrier()
    pltpu.sync_copy(deg_sh.at[pl.ds(s * STRIPE, STRIPE)],
                    out_hbm.at[c, pl.ds(s * STRIPE, STRIPE)])


@functools.partial(
    pl.kernel,
    out_type=jax.ShapeDtypeStruct((2, NPAD, D), jnp.float32),
    mesh=_mesh,
    scratch_types=[
        pltpu.VMEM((2, C), jnp.int32),
        pltpu.VMEM((CH, C), jnp.int32),
        pltpu.VMEM((2, C, D), jnp.float32),
        pltpu.VMEM_SHARED((NPAD, D), jnp.float32),
        pltpu.SemaphoreType.DMA,
        pltpu.SemaphoreType.DMA,
        pltpu.SemaphoreType.DMA,
        pltpu.SemaphoreType.DMA,
    ],
)
def _edge_kernel(g_hbm, src_hbm, dst_hbm, znd_hbm, out_hbm,
                 src_i, dst_all, rows, accum_sh, gsem, ssem, isem0, isem1):
    c = lax.axis_index("c")
    s = lax.axis_index("s")
    wid = s * 2 + c
    pltpu.sync_copy(znd_hbm.at[pl.ds(s * STRIPE, STRIPE)],
                    accum_sh.at[pl.ds(s * STRIPE, STRIPE)])
    pltpu.sync_copy(dst_hbm.at[wid], dst_all)
    plsc.subcore_barrier()

    isems = (isem0, isem1)

    def i_start(j, b):
        pltpu.async_copy(src_hbm.at[wid, pl.ds(j * C, C)], src_i.at[b],
                         isems[b])

    def i_wait(b):
        pltpu.make_async_copy(src_hbm.at[0, pl.ds(0, C)],
                              src_i.at[b], isems[b]).wait()

    def g_start(b):
        pltpu.async_copy(g_hbm.at[src_i.at[b]], rows.at[b], gsem)

    def g_wait(b):
        pltpu.make_async_copy(g_hbm.at[pl.ds(0, C)], rows.at[b], gsem).wait()

    def s_start(j, b):
        pltpu.async_copy(rows.at[b], accum_sh.at[dst_all.at[j]],
                         ssem, add=True)

    def s_wait(b):
        pltpu.make_async_copy(rows.at[b], accum_sh.at[pl.ds(0, C)],
                              ssem).wait()

    i_start(0, 0)
    i_wait(0)
    g_start(0)
    i_start(1, 1)
    g_wait(0)
    i_start(2, 0)
    i_wait(1)
    g_start(1)
    s_start(0, 0)
    g_wait(1)
    i_start(3, 1)
    s_wait(0)
    i_wait(0)
    g_start(0)
    s_start(1, 1)

    def steady(t, carry):
        j = 2 * t
        g_wait(0)
        i_start(j + 2, 0)
        s_wait(1)
        i_wait(1)
        g_start(1)
        s_start(j, 0)
        g_wait(1)
        i_start(j + 3, 1)
        s_wait(0)
        i_wait(0)
        g_start(0)
        s_start(j + 1, 1)
        return carry

    lax.fori_loop(1, CH // 2 - 1, steady, 0)

    g_wait(0)
    s_wait(1)
    i_wait(1)
    g_start(1)
    s_start(CH - 2, 0)
    g_wait(1)
    s_wait(0)
    s_start(CH - 1, 1)
    s_wait(1)

    plsc.subcore_barrier()
    pltpu.sync_copy(accum_sh.at[pl.ds(s * STRIPE, STRIPE)],
                    out_hbm.at[c, pl.ds(s * STRIPE, STRIPE)])


BR = 512


def _k1_body(degT_ref, x_ref, w_ref, dinv_ref, g_ref):
    deg = degT_ref[:, 0:1] + degT_ref[:, 1:2] + 1.0
    dinv = lax.rsqrt(deg)
    dinv_ref[...] = dinv
    h = jnp.dot(x_ref[...], w_ref[...], preferred_element_type=jnp.float32)
    g_ref[...] = h * dinv


def _k1(degT, x_pad, w):
    return pl.pallas_call(
        _k1_body,
        grid=(NPAD // BR,),
        in_specs=[
            pl.BlockSpec((BR, 2), lambda i: (i, 0)),
            pl.BlockSpec((BR, D), lambda i: (i, 0)),
            pl.BlockSpec((D, D), lambda i: (0, 0)),
        ],
        out_specs=[
            pl.BlockSpec((BR, 1), lambda i: (i, 0)),
            pl.BlockSpec((BR, D), lambda i: (i, 0)),
        ],
        out_shape=[
            jax.ShapeDtypeStruct((NPAD, 1), jnp.float32),
            jax.ShapeDtypeStruct((NPAD, D), jnp.float32),
        ],
    )(degT, x_pad, w)


def _k2_body(p_ref, g_ref, dinv_ref, b_ref, w_ref, gout_ref):
    a = p_ref[0] + p_ref[1] + g_ref[...]
    y = jnp.maximum(a * dinv_ref[...] + b_ref[...], 0.0)
    gout_ref[...] = jnp.dot(y, w_ref[...],
                            preferred_element_type=jnp.float32) * dinv_ref[...]


def _k2(p, g, dinv, b, w):
    return pl.pallas_call(
        _k2_body,
        grid=(NPAD // BR,),
        in_specs=[
            pl.BlockSpec((2, BR, D), lambda i: (0, i, 0)),
            pl.BlockSpec((BR, D), lambda i: (i, 0)),
            pl.BlockSpec((BR, 1), lambda i: (i, 0)),
            pl.BlockSpec((1, D), lambda i: (0, 0)),
            pl.BlockSpec((D, D), lambda i: (0, 0)),
        ],
        out_specs=pl.BlockSpec((BR, D), lambda i: (i, 0)),
        out_shape=jax.ShapeDtypeStruct((NPAD, D), jnp.float32),
    )(p, g, dinv, b, w)


def _k3_body(p_ref, g_ref, dinv_ref, b_ref, y_ref):
    a = p_ref[0] + p_ref[1] + g_ref[...]
    y_ref[...] = jnp.maximum(a * dinv_ref[...] + b_ref[...], 0.0)


def _k3(p, g, dinv, b):
    return pl.pallas_call(
        _k3_body,
        grid=(NPAD // BR,),
        in_specs=[
            pl.BlockSpec((2, BR, D), lambda i: (0, i, 0)),
            pl.BlockSpec((BR, D), lambda i: (i, 0)),
            pl.BlockSpec((BR, 1), lambda i: (i, 0)),
            pl.BlockSpec((1, D), lambda i: (0, 0)),
        ],
        out_specs=pl.BlockSpec((BR, D), lambda i: (i, 0)),
        out_shape=jax.ShapeDtypeStruct((NPAD, D), jnp.float32),
    )(p, g, dinv, b)


def kernel(x, edge_index, W1, b1, W2, b2):
    src = edge_index[0]
    dst = edge_index[1]
    pad = EPAD - E
    fill = N + jnp.arange(pad, dtype=jnp.int32) % (NPAD - N)
    srcp = jnp.concatenate([src, fill]).reshape(NW, CH * C)
    dstp = jnp.concatenate([dst, fill]).reshape(NW, CH, C)
    dstd = jnp.concatenate([dst, fill]).reshape(NW, CHD, CD)
    x_pad = jnp.pad(x, ((0, NPAD - N), (0, 0)))
    zeros_nd = jnp.zeros((NPAD, D), jnp.float32)
    zeros_n = jnp.zeros((NPAD,), jnp.float32)
    ones_c = jnp.ones((CD,), jnp.float32)

    degp = _deg_kernel(dstd, ones_c, zeros_n)
    degT = degp.T
    b1r = b1.reshape(1, D)
    b2r = b2.reshape(1, D)

    dinv, g1 = _k1(degT, x_pad, W1)
    p1 = _edge_kernel(g1, srcp, dstp, zeros_nd)
    g2 = _k2(p1, g1, dinv, b1r, W2)
    p2 = _edge_kernel(g2, srcp, dstp, zeros_nd)
    y = _k3(p2, g2, dinv, b2r)
    return y[:N]


# speedup vs baseline: 3.4380x; 1.0001x over previous
import functools

import jax
import jax.numpy as jnp
from jax import lax
from jax.experimental import pallas as pl
from jax.experimental.pallas import tpu as pltpu
from jax.experimental.pallas import tpu_sc as plsc

N = 10000
E = 320000
D = 128

NPAD = 10240
NW = 32
STRIPE = NPAD // 16

C = 128
CH = 80
EPAD = NW * CH * C

CD = 128
CHD = EPAD // (NW * CD)

_mesh = plsc.VectorSubcoreMesh(core_axis_name="c", subcore_axis_name="s")


@functools.partial(
    pl.kernel,
    out_type=jax.ShapeDtypeStruct((2, NPAD), jnp.float32),
    mesh=_mesh,
    scratch_types=[
        pltpu.VMEM((CHD, CD), jnp.int32),
        pltpu.VMEM((CD,), jnp.float32),
        pltpu.VMEM_SHARED((NPAD,), jnp.float32),
        pltpu.SemaphoreType.DMA,
    ],
)
def _deg_kernel(dst_hbm, ones_hbm, zn_hbm, out_hbm, dst_all, ones_v, deg_sh, sem):
    c = lax.axis_index("c")
    s = lax.axis_index("s")
    wid = s * 2 + c
    pltpu.sync_copy(zn_hbm.at[pl.ds(s * STRIPE, STRIPE)],
                    deg_sh.at[pl.ds(s * STRIPE, STRIPE)])
    pltpu.sync_copy(dst_hbm.at[wid], dst_all)
    pltpu.sync_copy(ones_hbm, ones_v)
    plsc.subcore_barrier()

    def fire(j, carry):
        pltpu.async_copy(ones_v, deg_sh.at[dst_all.at[j]], sem, add=True)
        return carry

    def drain(j, carry):
        pltpu.make_async_copy(ones_v, deg_sh.at[pl.ds(0, CD)], sem).wait()
        return carry

    lax.fori_loop(0, CHD, fire, 0)
    lax.fori_loop(0, CHD, drain, 0)
    plsc.subcore_barrier()
    pltpu.sync_copy(deg_sh.at[pl.ds(s * STRIPE, STRIPE)],
                    out_hbm.at[c, pl.ds(s * STRIPE, STRIPE)])


@functools.partial(
    pl.kernel,
    out_type=jax.ShapeDtypeStruct((2, NPAD, D), jnp.float32),
    mesh=_mesh,
    scratch_types=[
        pltpu.VMEM((2, C), jnp.int32),
        pltpu.VMEM((CH, C), jnp.int32),
        pltpu.VMEM((2, C, D), jnp.float32),
        pltpu.VMEM_SHARED((NPAD, D), jnp.float32),
        pltpu.SemaphoreType.DMA,
        pltpu.SemaphoreType.DMA,
        pltpu.SemaphoreType.DMA,
        pltpu.SemaphoreType.DMA,
        pltpu.SemaphoreType.DMA,
    ],
)
def _edge_kernel(g_hbm, src_hbm, dst_hbm, znd_hbm, out_hbm,
                 src_i, dst_all, rows, accum_sh, gsem, ssem0, ssem1,
                 isem0, isem1):
    c = lax.axis_index("c")
    s = lax.axis_index("s")
    wid = s * 2 + c
    pltpu.sync_copy(znd_hbm.at[pl.ds(s * STRIPE, STRIPE)],
                    accum_sh.at[pl.ds(s * STRIPE, STRIPE)])
    pltpu.sync_copy(dst_hbm.at[wid], dst_all)
    plsc.subcore_barrier()

    isems = (isem0, isem1)
    ssems = (ssem0, ssem1)

    def i_start(j, b):
        pltpu.async_copy(src_hbm.at[wid, pl.ds(j * C, C)], src_i.at[b],
                         isems[b])

    def i_wait(b):
        pltpu.make_async_copy(src_hbm.at[0, pl.ds(0, C)],
                              src_i.at[b], isems[b]).wait()

    def g_start(b):
        pltpu.async_copy(g_hbm.at[src_i.at[b]], rows.at[b], gsem)

    def g_wait(b):
        pltpu.make_async_copy(g_hbm.at[pl.ds(0, C)], rows.at[b], gsem).wait()

    def s_start(j, b):
        pltpu.async_copy(rows.at[b], accum_sh.at[dst_all.at[j]],
                         ssems[b], add=True)

    def s_wait(b):
        pltpu.make_async_copy(rows.at[b], accum_sh.at[pl.ds(0, C)],
                              ssems[b]).wait()

    i_start(0, 0)
    i_wait(0)
    g_start(0)
    i_start(1, 1)
    g_wait(0)
    i_start(2, 0)
    i_wait(1)
    g_start(1)
    s_start(0, 0)
    g_wait(1)
    i_start(3, 1)
    s_start(1, 1)
    s_wait(0)
    i_wait(0)
    g_start(0)

    def steady(t, carry):
        j = 2 * t
        g_wait(0)
        i_start(j + 2, 0)
        s_wait(1)
        i_wait(1)
        g_start(1)
        s_start(j, 0)
        g_wait(1)
        i_start(j + 3, 1)
        s_start(j + 1, 1)
        s_wait(0)
        i_wait(0)
        g_start(0)
        return carry

    lax.fori_loop(1, CH // 2 - 1, steady, 0)

    g_wait(0)
    s_wait(1)
    i_wait(1)
    g_start(1)
    s_start(CH - 2, 0)
    g_wait(1)
    s_start(CH - 1, 1)
    s_wait(0)
    s_wait(1)

    plsc.subcore_barrier()
    pltpu.sync_copy(accum_sh.at[pl.ds(s * STRIPE, STRIPE)],
                    out_hbm.at[c, pl.ds(s * STRIPE, STRIPE)])


BR = 512


def _k1_body(degT_ref, x_ref, w_ref, dinv_ref, g_ref):
    deg = degT_ref[:, 0:1] + degT_ref[:, 1:2] + 1.0
    dinv = lax.rsqrt(deg)
    dinv_ref[...] = dinv
    h = jnp.dot(x_ref[...], w_ref[...], preferred_element_type=jnp.float32)
    g_ref[...] = h * dinv


def _k1(degT, x_pad, w):
    return pl.pallas_call(
        _k1_body,
        grid=(NPAD // BR,),
        in_specs=[
            pl.BlockSpec((BR, 2), lambda i: (i, 0)),
            pl.BlockSpec((BR, D), lambda i: (i, 0)),
            pl.BlockSpec((D, D), lambda i: (0, 0)),
        ],
        out_specs=[
            pl.BlockSpec((BR, 1), lambda i: (i, 0)),
            pl.BlockSpec((BR, D), lambda i: (i, 0)),
        ],
        out_shape=[
            jax.ShapeDtypeStruct((NPAD, 1), jnp.float32),
            jax.ShapeDtypeStruct((NPAD, D), jnp.float32),
        ],
    )(degT, x_pad, w)


def _k2_body(p_ref, g_ref, dinv_ref, b_ref, w_ref, gout_ref):
    a = p_ref[0] + p_ref[1] + g_ref[...]
    y = jnp.maximum(a * dinv_ref[...] + b_ref[...], 0.0)
    gout_ref[...] = jnp.dot(y, w_ref[...],
                            preferred_element_type=jnp.float32) * dinv_ref[...]


def _k2(p, g, dinv, b, w):
    return pl.pallas_call(
        _k2_body,
        grid=(NPAD // BR,),
        in_specs=[
            pl.BlockSpec((2, BR, D), lambda i: (0, i, 0)),
            pl.BlockSpec((BR, D), lambda i: (i, 0)),
            pl.BlockSpec((BR, 1), lambda i: (i, 0)),
            pl.BlockSpec((1, D), lambda i: (0, 0)),
            pl.BlockSpec((D, D), lambda i: (0, 0)),
        ],
        out_specs=pl.BlockSpec((BR, D), lambda i: (i, 0)),
        out_shape=jax.ShapeDtypeStruct((NPAD, D), jnp.float32),
    )(p, g, dinv, b, w)


def _k3_body(p_ref, g_ref, dinv_ref, b_ref, y_ref):
    a = p_ref[0] + p_ref[1] + g_ref[...]
    y_ref[...] = jnp.maximum(a * dinv_ref[...] + b_ref[...], 0.0)


def _k3(p, g, dinv, b):
    return pl.pallas_call(
        _k3_body,
        grid=(NPAD // BR,),
        in_specs=[
            pl.BlockSpec((2, BR, D), lambda i: (0, i, 0)),
            pl.BlockSpec((BR, D), lambda i: (i, 0)),
            pl.BlockSpec((BR, 1), lambda i: (i, 0)),
            pl.BlockSpec((1, D), lambda i: (0, 0)),
        ],
        out_specs=pl.BlockSpec((BR, D), lambda i: (i, 0)),
        out_shape=jax.ShapeDtypeStruct((NPAD, D), jnp.float32),
    )(p, g, dinv, b)


def kernel(x, edge_index, W1, b1, W2, b2):
    src = edge_index[0]
    dst = edge_index[1]
    pad = EPAD - E
    fill = N + jnp.arange(pad, dtype=jnp.int32) % (NPAD - N)
    srcp = jnp.concatenate([src, fill]).reshape(NW, CH * C)
    dstp = jnp.concatenate([dst, fill]).reshape(NW, CH, C)
    dstd = jnp.concatenate([dst, fill]).reshape(NW, CHD, CD)
    x_pad = jnp.pad(x, ((0, NPAD - N), (0, 0)))
    zeros_nd = jnp.zeros((NPAD, D), jnp.float32)
    zeros_n = jnp.zeros((NPAD,), jnp.float32)
    ones_c = jnp.ones((CD,), jnp.float32)

    degp = _deg_kernel(dstd, ones_c, zeros_n)
    degT = degp.T
    b1r = b1.reshape(1, D)
    b2r = b2.reshape(1, D)

    dinv, g1 = _k1(degT, x_pad, W1)
    p1 = _edge_kernel(g1, srcp, dstp, zeros_nd)
    g2 = _k2(p1, g1, dinv, b1r, W2)
    p2 = _edge_kernel(g2, srcp, dstp, zeros_nd)
    y = _k3(p2, g2, dinv, b2r)
    return y[:N]


# speedup vs baseline: 3.5121x; 1.0216x over previous
import functools

import jax
import jax.numpy as jnp
from jax import lax
from jax.experimental import pallas as pl
from jax.experimental.pallas import tpu as pltpu
from jax.experimental.pallas import tpu_sc as plsc

N = 10000
E = 320000
D = 128

NPAD = 10240
NW = 32
STRIPE = NPAD // 16

C = 128
CH = 80
EPAD = NW * CH * C

CD = 128
CHD = EPAD // (NW * CD)

_mesh = plsc.VectorSubcoreMesh(core_axis_name="c", subcore_axis_name="s")


@functools.partial(
    pl.kernel,
    out_type=jax.ShapeDtypeStruct((2, NPAD), jnp.float32),
    mesh=_mesh,
    scratch_types=[
        pltpu.VMEM((CHD, CD), jnp.int32),
        pltpu.VMEM((CD,), jnp.float32),
        pltpu.VMEM_SHARED((NPAD,), jnp.float32),
        pltpu.SemaphoreType.DMA,
    ],
)
def _deg_kernel(dst_hbm, ones_hbm, zn_hbm, out_hbm, dst_all, ones_v, deg_sh, sem):
    c = lax.axis_index("c")
    s = lax.axis_index("s")
    wid = s * 2 + c
    pltpu.sync_copy(zn_hbm.at[pl.ds(s * STRIPE, STRIPE)],
                    deg_sh.at[pl.ds(s * STRIPE, STRIPE)])
    pltpu.sync_copy(dst_hbm.at[wid], dst_all)
    pltpu.sync_copy(ones_hbm, ones_v)
    plsc.subcore_barrier()

    def fire(j, carry):
        pltpu.async_copy(ones_v, deg_sh.at[dst_all.at[j]], sem, add=True)
        return carry

    def drain(j, carry):
        pltpu.make_async_copy(ones_v, deg_sh.at[pl.ds(0, CD)], sem).wait()
        return carry

    lax.fori_loop(0, CHD, fire, 0)
    lax.fori_loop(0, CHD, drain, 0)
    plsc.subcore_barrier()
    pltpu.sync_copy(deg_sh.at[pl.ds(s * STRIPE, STRIPE)],
                    out_hbm.at[c, pl.ds(s * STRIPE, STRIPE)])


@functools.partial(
    pl.kernel,
    out_type=jax.ShapeDtypeStruct((2, NPAD, D), jnp.float32),
    mesh=_mesh,
    scratch_types=[
        pltpu.VMEM((2, C), jnp.int32),
        pltpu.VMEM((CH, C), jnp.int32),
        pltpu.VMEM((2, C, D), jnp.float32),
        pltpu.VMEM_SHARED((NPAD, D), jnp.float32),
        pltpu.SemaphoreType.DMA,
        pltpu.SemaphoreType.DMA,
        pltpu.SemaphoreType.DMA,
        pltpu.SemaphoreType.DMA,
        pltpu.SemaphoreType.DMA,
    ],
)
def _edge_kernel(g_hbm, src_hbm, dst_hbm, znd_hbm, out_hbm,
                 src_i, dst_all, rows, accum_sh, gsem, ssem0, ssem1,
                 isem0, isem1):
    c = lax.axis_index("c")
    s = lax.axis_index("s")
    wid = s * 2 + c
    pltpu.sync_copy(znd_hbm.at[pl.ds(s * STRIPE, STRIPE)],
                    accum_sh.at[pl.ds(s * STRIPE, STRIPE)])
    pltpu.sync_copy(dst_hbm.at[wid], dst_all)
    plsc.subcore_barrier()

    isems = (isem0, isem1)
    ssems = (ssem0, ssem1)

    def i_start(j, b):
        pltpu.async_copy(src_hbm.at[wid, pl.ds(j * C, C)], src_i.at[b],
                         isems[b])

    def i_wait(b):
        pltpu.make_async_copy(src_hbm.at[0, pl.ds(0, C)],
                              src_i.at[b], isems[b]).wait()

    def g_start(b):
        pltpu.async_copy(g_hbm.at[src_i.at[b]], rows.at[b], gsem)

    def g_wait(b):
        pltpu.make_async_copy(g_hbm.at[pl.ds(0, C)], rows.at[b], gsem).wait()

    def s_start(j, b):
        pltpu.async_copy(rows.at[b], accum_sh.at[dst_all.at[j]],
                         ssems[b], add=True)

    def s_wait(b):
        pltpu.make_async_copy(rows.at[b], accum_sh.at[pl.ds(0, C)],
                              ssems[b]).wait()

    i_start(0, 0)
    i_wait(0)
    g_start(0)
    i_start(1, 1)
    g_wait(0)
    i_start(2, 0)
    i_wait(1)
    g_start(1)
    s_start(0, 0)
    g_wait(1)
    i_start(3, 1)
    s_start(1, 1)
    s_wait(0)
    i_wait(0)
    g_start(0)

    def steady(t, carry):
        j = 2 * t
        g_wait(0)
        i_start(j + 2, 0)
        s_wait(1)
        i_wait(1)
        g_start(1)
        s_start(j, 0)
        g_wait(1)
        i_start(j + 3, 1)
        s_start(j + 1, 1)
        s_wait(0)
        i_wait(0)
        g_start(0)
        return carry

    lax.fori_loop(1, CH // 2 - 1, steady, 0)

    g_wait(0)
    s_wait(1)
    i_wait(1)
    g_start(1)
    s_start(CH - 2, 0)
    g_wait(1)
    s_start(CH - 1, 1)
    s_wait(0)
    s_wait(1)

    plsc.subcore_barrier()
    pltpu.sync_copy(accum_sh.at[pl.ds(s * STRIPE, STRIPE)],
                    out_hbm.at[c, pl.ds(s * STRIPE, STRIPE)])


BR = 1024


def _k1_body(degT_ref, x_ref, w_ref, dinv_ref, g_ref):
    deg = degT_ref[:, 0:1] + degT_ref[:, 1:2] + 1.0
    dinv = lax.rsqrt(deg)
    dinv_ref[...] = dinv
    h = jnp.dot(x_ref[...], w_ref[...], preferred_element_type=jnp.float32)
    g_ref[...] = h * dinv


def _k1(degT, x_pad, w):
    return pl.pallas_call(
        _k1_body,
        grid=(NPAD // BR,),
        in_specs=[
            pl.BlockSpec((BR, 2), lambda i: (i, 0)),
            pl.BlockSpec((BR, D), lambda i: (i, 0)),
            pl.BlockSpec((D, D), lambda i: (0, 0)),
        ],
        out_specs=[
            pl.BlockSpec((BR, 1), lambda i: (i, 0)),
            pl.BlockSpec((BR, D), lambda i: (i, 0)),
        ],
        out_shape=[
            jax.ShapeDtypeStruct((NPAD, 1), jnp.float32),
            jax.ShapeDtypeStruct((NPAD, D), jnp.float32),
        ],
    )(degT, x_pad, w)


def _k2_body(p_ref, g_ref, dinv_ref, b_ref, w_ref, gout_ref):
    a = p_ref[0] + p_ref[1] + g_ref[...]
    y = jnp.maximum(a * dinv_ref[...] + b_ref[...], 0.0)
    gout_ref[...] = jnp.dot(y, w_ref[...],
                            preferred_element_type=jnp.float32) * dinv_ref[...]


def _k2(p, g, dinv, b, w):
    return pl.pallas_call(
        _k2_body,
        grid=(NPAD // BR,),
        in_specs=[
            pl.BlockSpec((2, BR, D), lambda i: (0, i, 0)),
            pl.BlockSpec((BR, D), lambda i: (i, 0)),
            pl.BlockSpec((BR, 1), lambda i: (i, 0)),
            pl.BlockSpec((1, D), lambda i: (0, 0)),
            pl.BlockSpec((D, D), lambda i: (0, 0)),
        ],
        out_specs=pl.BlockSpec((BR, D), lambda i: (i, 0)),
        out_shape=jax.ShapeDtypeStruct((NPAD, D), jnp.float32),
    )(p, g, dinv, b, w)


def _k3_body(p_ref, g_ref, dinv_ref, b_ref, y_ref):
    a = p_ref[0] + p_ref[1] + g_ref[...]
    y_ref[...] = jnp.maximum(a * dinv_ref[...] + b_ref[...], 0.0)


def _k3(p, g, dinv, b):
    return pl.pallas_call(
        _k3_body,
        grid=(NPAD // BR,),
        in_specs=[
            pl.BlockSpec((2, BR, D), lambda i: (0, i, 0)),
            pl.BlockSpec((BR, D), lambda i: (i, 0)),
            pl.BlockSpec((BR, 1), lambda i: (i, 0)),
            pl.BlockSpec((1, D), lambda i: (0, 0)),
        ],
        out_specs=pl.BlockSpec((BR, D), lambda i: (i, 0)),
        out_shape=jax.ShapeDtypeStruct((NPAD, D), jnp.float32),
    )(p, g, dinv, b)


def kernel(x, edge_index, W1, b1, W2, b2):
    pad = EPAD - E
    fill = N + jnp.arange(pad, dtype=jnp.int32) % (NPAD - N)
    eip = jnp.concatenate(
        [edge_index, jnp.broadcast_to(fill, (2, pad))], axis=1)
    srcp = eip[0].reshape(NW, CH * C)
    dstp = eip[1].reshape(NW, CH, C)
    dstd = eip[1].reshape(NW, CHD, CD)
    x_pad = jnp.pad(x, ((0, NPAD - N), (0, 0)))
    zeros_nd = jnp.zeros((NPAD, D), jnp.float32)
    zeros_n = jnp.zeros((NPAD,), jnp.float32)
    ones_c = jnp.ones((CD,), jnp.float32)

    degp = _deg_kernel(dstd, ones_c, zeros_n)
    degT = degp.T
    b1r = b1.reshape(1, D)
    b2r = b2.reshape(1, D)

    dinv, g1 = _k1(degT, x_pad, W1)
    p1 = _edge_kernel(g1, srcp, dstp, zeros_nd)
    g2 = _k2(p1, g1, dinv, b1r, W2)
    p2 = _edge_kernel(g2, srcp, dstp, zeros_nd)
    y = _k3(p2, g2, dinv, b2r)
    return y[:N]


# speedup vs baseline: 3.5140x; 1.0005x over previous
import functools

import jax
import jax.numpy as jnp
from jax import lax
from jax.experimental import pallas as pl
from jax.experimental.pallas import tpu as pltpu
from jax.experimental.pallas import tpu_sc as plsc

N = 10000
E = 320000
D = 128

NPAD = 10240
NW = 32
STRIPE = NPAD // 16

C = 128
CH = 80
EPAD = NW * CH * C

CD = 128
CHD = EPAD // (NW * CD)

_mesh = plsc.VectorSubcoreMesh(core_axis_name="c", subcore_axis_name="s")


@functools.partial(
    pl.kernel,
    out_type=jax.ShapeDtypeStruct((2, NPAD), jnp.float32),
    mesh=_mesh,
    scratch_types=[
        pltpu.VMEM((CHD, CD), jnp.int32),
        pltpu.VMEM((CD,), jnp.float32),
        pltpu.VMEM_SHARED((NPAD,), jnp.float32),
        pltpu.SemaphoreType.DMA,
    ],
)
def _deg_kernel(dst_hbm, ones_hbm, zn_hbm, out_hbm, dst_all, ones_v, deg_sh, sem):
    c = lax.axis_index("c")
    s = lax.axis_index("s")
    wid = s * 2 + c
    pltpu.sync_copy(zn_hbm.at[pl.ds(s * STRIPE, STRIPE)],
                    deg_sh.at[pl.ds(s * STRIPE, STRIPE)])
    pltpu.sync_copy(dst_hbm.at[wid], dst_all)
    pltpu.sync_copy(ones_hbm, ones_v)
    plsc.subcore_barrier()

    def fire(j, carry):
        pltpu.async_copy(ones_v, deg_sh.at[dst_all.at[j]], sem, add=True)
        return carry

    def drain(j, carry):
        pltpu.make_async_copy(ones_v, deg_sh.at[pl.ds(0, CD)], sem).wait()
        return carry

    lax.fori_loop(0, CHD, fire, 0)
    lax.fori_loop(0, CHD, drain, 0)
    plsc.subcore_barrier()
    pltpu.sync_copy(deg_sh.at[pl.ds(s * STRIPE, STRIPE)],
                    out_hbm.at[c, pl.ds(s * STRIPE, STRIPE)])


@functools.partial(
    pl.kernel,
    out_type=jax.ShapeDtypeStruct((2, NPAD, D), jnp.float32),
    mesh=_mesh,
    scratch_types=[
        pltpu.VMEM((2, C), jnp.int32),
        pltpu.VMEM((CH, C), jnp.int32),
        pltpu.VMEM((2, C, D), jnp.float32),
        pltpu.VMEM_SHARED((NPAD, D), jnp.float32),
        pltpu.SemaphoreType.DMA,
        pltpu.SemaphoreType.DMA,
        pltpu.SemaphoreType.DMA,
        pltpu.SemaphoreType.DMA,
        pltpu.SemaphoreType.DMA,
    ],
)
def _edge_kernel(g_hbm, src_hbm, dst_hbm, znd_hbm, out_hbm,
                 src_i, dst_all, rows, accum_sh, gsem, ssem0, ssem1,
                 isem0, isem1):
    c = lax.axis_index("c")
    s = lax.axis_index("s")
    wid = s * 2 + c
    pltpu.sync_copy(znd_hbm.at[pl.ds(s * STRIPE, STRIPE)],
                    accum_sh.at[pl.ds(s * STRIPE, STRIPE)])
    pltpu.sync_copy(dst_hbm.at[wid], dst_all)
    plsc.subcore_barrier()

    isems = (isem0, isem1)
    ssems = (ssem0, ssem1)

    def i_start(j, b):
        pltpu.async_copy(src_hbm.at[wid, pl.ds(j * C, C)], src_i.at[b],
                         isems[b])

    def i_wait(b):
        pltpu.make_async_copy(src_hbm.at[0, pl.ds(0, C)],
                              src_i.at[b], isems[b]).wait()

    def g_start(b):
        pltpu.async_copy(g_hbm.at[src_i.at[b]], rows.at[b], gsem)

    def g_wait(b):
        pltpu.make_async_copy(g_hbm.at[pl.ds(0, C)], rows.at[b], gsem).wait()

    def s_start(j, b):
        pltpu.async_copy(rows.at[b], accum_sh.at[dst_all.at[j]],
                         ssems[b], add=True)

    def s_wait(b):
        pltpu.make_async_copy(rows.at[b], accum_sh.at[pl.ds(0, C)],
                              ssems[b]).wait()

    i_start(0, 0)
    i_wait(0)
    g_start(0)
    i_start(1, 1)
    g_wait(0)
    i_start(2, 0)
    i_wait(1)
    g_start(1)
    s_start(0, 0)
    g_wait(1)
    i_start(3, 1)
    s_start(1, 1)
    s_wait(0)
    i_wait(0)
    g_start(0)

    def steady(t, carry):
        j = 2 * t
        g_wait(0)
        i_start(j + 2, 0)
        s_wait(1)
        i_wait(1)
        g_start(1)
        s_start(j, 0)
        g_wait(1)
        i_start(j + 3, 1)
        s_start(j + 1, 1)
        s_wait(0)
        i_wait(0)
        g_start(0)
        return carry

    lax.fori_loop(1, CH // 2 - 1, steady, 0)

    g_wait(0)
    s_wait(1)
    i_wait(1)
    g_start(1)
    s_start(CH - 2, 0)
    g_wait(1)
    s_start(CH - 1, 1)
    s_wait(0)
    s_wait(1)

    plsc.subcore_barrier()
    pltpu.sync_copy(accum_sh.at[pl.ds(s * STRIPE, STRIPE)],
                    out_hbm.at[c, pl.ds(s * STRIPE, STRIPE)])


BR = 1024


def _k0_body(x_ref, w_ref, h_ref):
    h_ref[...] = jnp.dot(x_ref[...], w_ref[...],
                         preferred_element_type=jnp.float32)


def _k0(x_pad, w):
    return pl.pallas_call(
        _k0_body,
        grid=(NPAD // BR,),
        in_specs=[
            pl.BlockSpec((BR, D), lambda i: (i, 0)),
            pl.BlockSpec((D, D), lambda i: (0, 0)),
        ],
        out_specs=pl.BlockSpec((BR, D), lambda i: (i, 0)),
        out_shape=jax.ShapeDtypeStruct((NPAD, D), jnp.float32),
    )(x_pad, w)


def _k1_body(degT_ref, h_ref, dinv_ref, g_ref):
    deg = degT_ref[:, 0:1] + degT_ref[:, 1:2] + 1.0
    dinv = lax.rsqrt(deg)
    dinv_ref[...] = dinv
    g_ref[...] = h_ref[...] * dinv


def _k1(degT, h):
    return pl.pallas_call(
        _k1_body,
        grid=(NPAD // BR,),
        in_specs=[
            pl.BlockSpec((BR, 2), lambda i: (i, 0)),
            pl.BlockSpec((BR, D), lambda i: (i, 0)),
        ],
        out_specs=[
            pl.BlockSpec((BR, 1), lambda i: (i, 0)),
            pl.BlockSpec((BR, D), lambda i: (i, 0)),
        ],
        out_shape=[
            jax.ShapeDtypeStruct((NPAD, 1), jnp.float32),
            jax.ShapeDtypeStruct((NPAD, D), jnp.float32),
        ],
    )(degT, h)


def _k2_body(p_ref, g_ref, dinv_ref, b_ref, w_ref, gout_ref):
    a = p_ref[0] + p_ref[1] + g_ref[...]
    y = jnp.maximum(a * dinv_ref[...] + b_ref[...], 0.0)
    gout_ref[...] = jnp.dot(y, w_ref[...],
                            preferred_element_type=jnp.float32) * dinv_ref[...]


def _k2(p, g, dinv, b, w):
    return pl.pallas_call(
        _k2_body,
        grid=(NPAD // BR,),
        in_specs=[
            pl.BlockSpec((2, BR, D), lambda i: (0, i, 0)),
            pl.BlockSpec((BR, D), lambda i: (i, 0)),
            pl.BlockSpec((BR, 1), lambda i: (i, 0)),
            pl.BlockSpec((1, D), lambda i: (0, 0)),
            pl.BlockSpec((D, D), lambda i: (0, 0)),
        ],
        out_specs=pl.BlockSpec((BR, D), lambda i: (i, 0)),
        out_shape=jax.ShapeDtypeStruct((NPAD, D), jnp.float32),
    )(p, g, dinv, b, w)


def _k3_body(p_ref, g_ref, dinv_ref, b_ref, y_ref):
    a = p_ref[0] + p_ref[1] + g_ref[...]
    y_ref[...] = jnp.maximum(a * dinv_ref[...] + b_ref[...], 0.0)


def _k3(p, g, dinv, b):
    return pl.pallas_call(
        _k3_body,
        grid=(pl.cdiv(N, BR),),
        in_specs=[
            pl.BlockSpec((2, BR, D), lambda i: (0, i, 0)),
            pl.BlockSpec((BR, D), lambda i: (i, 0)),
            pl.BlockSpec((BR, 1), lambda i: (i, 0)),
            pl.BlockSpec((1, D), lambda i: (0, 0)),
        ],
        out_specs=pl.BlockSpec((BR, D), lambda i: (i, 0)),
        out_shape=jax.ShapeDtypeStruct((N, D), jnp.float32),
    )(p, g, dinv, b)


def kernel(x, edge_index, W1, b1, W2, b2):
    pad = EPAD - E
    fill = N + jnp.arange(pad, dtype=jnp.int32) % (NPAD - N)
    eip = jnp.concatenate(
        [edge_index, jnp.broadcast_to(fill, (2, pad))], axis=1)
    srcp = eip[0].reshape(NW, CH * C)
    dstp = eip[1].reshape(NW, CH, C)
    dstd = eip[1].reshape(NW, CHD, CD)
    x_pad = jnp.pad(x, ((0, NPAD - N), (0, 0)))
    zeros_nd = jnp.zeros((NPAD, D), jnp.float32)
    zeros_n = jnp.zeros((NPAD,), jnp.float32)
    ones_c = jnp.ones((CD,), jnp.float32)

    degp = _deg_kernel(dstd, ones_c, zeros_n)
    h1 = _k0(x_pad, W1)
    degT = degp.T
    b1r = b1.reshape(1, D)
    b2r = b2.reshape(1, D)

    dinv, g1 = _k1(degT, h1)
    p1 = _edge_kernel(g1, srcp, dstp, zeros_nd)
    g2 = _k2(p1, g1, dinv, b1r, W2)
    p2 = _edge_kernel(g2, srcp, dstp, zeros_nd)
    return _k3(p2, g2, dinv, b2r)


# speedup vs baseline: 3.6327x; 1.0338x over previous
import functools

import jax
import jax.numpy as jnp
from jax import lax
from jax.experimental import pallas as pl
from jax.experimental.pallas import tpu as pltpu
from jax.experimental.pallas import tpu_sc as plsc

N = 10000
E = 320000
D = 128

NPAD = 10240
NW = 32
STRIPE = NPAD // 16

C = 128
CH = 80
EPAD = NW * CH * C

CD = 128
CHD = EPAD // (NW * CD)

ROWS_E = E // C
TILE_CH = CH
PAD_ROWS = NW * TILE_CH - ROWS_E
LAST_REAL = ROWS_E - (NW - 1) * TILE_CH

_mesh = plsc.VectorSubcoreMesh(core_axis_name="c", subcore_axis_name="s")


def _load_chunk_rows(wid, real_hbm, pad_hbm, dst_vmem, sem):
    def fire(j, carry):
        r = wid * TILE_CH + j

        @pl.when(r < ROWS_E)
        def _():
            pltpu.async_copy(real_hbm.at[r], dst_vmem.at[j], sem)

        @pl.when(r >= ROWS_E)
        def _():
            pltpu.async_copy(pad_hbm.at[r - ROWS_E], dst_vmem.at[j], sem)

        return carry

    def drain(j, carry):
        pltpu.make_async_copy(pad_hbm.at[0], dst_vmem.at[0], sem).wait()
        return carry

    lax.fori_loop(0, TILE_CH, fire, 0)
    lax.fori_loop(0, TILE_CH, drain, 0)


@functools.partial(
    pl.kernel,
    out_type=jax.ShapeDtypeStruct((2, NPAD), jnp.float32),
    mesh=_mesh,
    scratch_types=[
        pltpu.VMEM((CHD, CD), jnp.int32),
        pltpu.VMEM((CD,), jnp.float32),
        pltpu.VMEM_SHARED((NPAD,), jnp.float32),
        pltpu.SemaphoreType.DMA,
    ],
)
def _deg_kernel(dst_hbm, pad_hbm, ones_hbm, zn_hbm, out_hbm,
                dst_all, ones_v, deg_sh, sem):
    c = lax.axis_index("c")
    s = lax.axis_index("s")
    wid = s * 2 + c
    pltpu.sync_copy(zn_hbm.at[pl.ds(s * STRIPE, STRIPE)],
                    deg_sh.at[pl.ds(s * STRIPE, STRIPE)])
    _load_chunk_rows(wid, dst_hbm, pad_hbm, dst_all, sem)
    pltpu.sync_copy(ones_hbm, ones_v)
    plsc.subcore_barrier()

    def fire(j, carry):
        pltpu.async_copy(ones_v, deg_sh.at[dst_all.at[j]], sem, add=True)
        return carry

    def drain(j, carry):
        pltpu.make_async_copy(ones_v, deg_sh.at[pl.ds(0, CD)], sem).wait()
        return carry

    lax.fori_loop(0, CHD, fire, 0)
    lax.fori_loop(0, CHD, drain, 0)
    plsc.subcore_barrier()
    pltpu.sync_copy(deg_sh.at[pl.ds(s * STRIPE, STRIPE)],
                    out_hbm.at[c, pl.ds(s * STRIPE, STRIPE)])


@functools.partial(
    pl.kernel,
    out_type=jax.ShapeDtypeStruct((2, NPAD, D), jnp.float32),
    mesh=_mesh,
    scratch_types=[
        pltpu.VMEM((2, C), jnp.int32),
        pltpu.VMEM((CH, C), jnp.int32),
        pltpu.VMEM((2, C, D), jnp.float32),
        pltpu.VMEM_SHARED((NPAD, D), jnp.float32),
        pltpu.SemaphoreType.DMA,
        pltpu.SemaphoreType.DMA,
        pltpu.SemaphoreType.DMA,
        pltpu.SemaphoreType.DMA,
        pltpu.SemaphoreType.DMA,
    ],
)
def _edge_kernel(g_hbm, src_hbm, dst_hbm, pad_hbm, znd_hbm, out_hbm,
                 src_i, dst_all, rows, accum_sh, gsem, ssem0, ssem1,
                 isem0, isem1):
    c = lax.axis_index("c")
    s = lax.axis_index("s")
    wid = s * 2 + c
    pltpu.sync_copy(znd_hbm.at[pl.ds(s * STRIPE, STRIPE)],
                    accum_sh.at[pl.ds(s * STRIPE, STRIPE)])
    _load_chunk_rows(wid, dst_hbm, pad_hbm, dst_all, gsem)
    plsc.subcore_barrier()

    isems = (isem0, isem1)
    ssems = (ssem0, ssem1)

    def i_start(j, b):
        r = wid * TILE_CH + j

        @pl.when(r < ROWS_E)
        def _():
            pltpu.async_copy(src_hbm.at[r], src_i.at[b], isems[b])

        @pl.when(r >= ROWS_E)
        def _():
            pltpu.async_copy(pad_hbm.at[r - ROWS_E], src_i.at[b], isems[b])

    def i_wait(b):
        pltpu.make_async_copy(pad_hbm.at[0], src_i.at[b], isems[b]).wait()

    def g_start(b):
        pltpu.async_copy(g_hbm.at[src_i.at[b]], rows.at[b], gsem)

    def g_wait(b):
        pltpu.make_async_copy(g_hbm.at[pl.ds(0, C)], rows.at[b], gsem).wait()

    def s_start(j, b):
        pltpu.async_copy(rows.at[b], accum_sh.at[dst_all.at[j]],
                         ssems[b], add=True)

    def s_wait(b):
        pltpu.make_async_copy(rows.at[b], accum_sh.at[pl.ds(0, C)],
                              ssems[b]).wait()

    i_start(0, 0)
    i_wait(0)
    g_start(0)
    i_start(1, 1)
    g_wait(0)
    i_start(2, 0)
    i_wait(1)
    g_start(1)
    s_start(0, 0)
    g_wait(1)
    i_start(3, 1)
    s_start(1, 1)
    s_wait(0)
    i_wait(0)
    g_start(0)

    def steady(t, carry):
        j = 2 * t
        g_wait(0)
        i_start(j + 2, 0)
        s_wait(1)
        i_wait(1)
        g_start(1)
        s_start(j, 0)
        g_wait(1)
        i_start(j + 3, 1)
        s_start(j + 1, 1)
        s_wait(0)
        i_wait(0)
        g_start(0)
        return carry

    lax.fori_loop(1, CH // 2 - 1, steady, 0)

    g_wait(0)
    s_wait(1)
    i_wait(1)
    g_start(1)
    s_start(CH - 2, 0)
    g_wait(1)
    s_start(CH - 1, 1)
    s_wait(0)
    s_wait(1)

    plsc.subcore_barrier()
    pltpu.sync_copy(accum_sh.at[pl.ds(s * STRIPE, STRIPE)],
                    out_hbm.at[c, pl.ds(s * STRIPE, STRIPE)])


BR = 1024


def _k0_body(x_ref, w_ref, h_ref):
    h_ref[...] = jnp.dot(x_ref[...], w_ref[...],
                         preferred_element_type=jnp.float32)


def _k0(x, w):
    return pl.pallas_call(
        _k0_body,
        grid=(NPAD // BR,),
        in_specs=[
            pl.BlockSpec((BR, D), lambda i: (i, 0)),
            pl.BlockSpec((D, D), lambda i: (0, 0)),
        ],
        out_specs=pl.BlockSpec((BR, D), lambda i: (i, 0)),
        out_shape=jax.ShapeDtypeStruct((NPAD, D), jnp.float32),
    )(x, w)


def _k1_body(degT_ref, h_ref, dinv_ref, g_ref):
    i = pl.program_id(0)
    rows = lax.broadcasted_iota(jnp.int32, (BR, 1), 0) + i * BR
    deg = degT_ref[:, 0:1] + degT_ref[:, 1:2] + 1.0
    dinv = lax.rsqrt(deg)
    dinv_ref[...] = dinv
    g_ref[...] = jnp.where(rows < N, h_ref[...] * dinv, 0.0)


def _k1(degT, h):
    return pl.pallas_call(
        _k1_body,
        grid=(NPAD // BR,),
        in_specs=[
            pl.BlockSpec((BR, 2), lambda i: (i, 0)),
            pl.BlockSpec((BR, D), lambda i: (i, 0)),
        ],
        out_specs=[
            pl.BlockSpec((BR, 1), lambda i: (i, 0)),
            pl.BlockSpec((BR, D), lambda i: (i, 0)),
        ],
        out_shape=[
            jax.ShapeDtypeStruct((NPAD, 1), jnp.float32),
            jax.ShapeDtypeStruct((NPAD, D), jnp.float32),
        ],
    )(degT, h)


def _k2_body(p_ref, g_ref, dinv_ref, b_ref, w_ref, gout_ref):
    i = pl.program_id(0)
    rows = lax.broadcasted_iota(jnp.int32, (BR, 1), 0) + i * BR
    a = p_ref[0] + p_ref[1] + g_ref[...]
    y = jnp.maximum(a * dinv_ref[...] + b_ref[...], 0.0)
    gout_ref[...] = jnp.where(
        rows < N,
        jnp.dot(y, w_ref[...],
                preferred_element_type=jnp.float32) * dinv_ref[...],
        0.0)


def _k2(p, g, dinv, b, w):
    return pl.pallas_call(
        _k2_body,
        grid=(NPAD // BR,),
        in_specs=[
            pl.BlockSpec((2, BR, D), lambda i: (0, i, 0)),
            pl.BlockSpec((BR, D), lambda i: (i, 0)),
            pl.BlockSpec((BR, 1), lambda i: (i, 0)),
            pl.BlockSpec((1, D), lambda i: (0, 0)),
            pl.BlockSpec((D, D), lambda i: (0, 0)),
        ],
        out_specs=pl.BlockSpec((BR, D), lambda i: (i, 0)),
        out_shape=jax.ShapeDtypeStruct((NPAD, D), jnp.float32),
    )(p, g, dinv, b, w)


def _k3_body(p_ref, g_ref, dinv_ref, b_ref, y_ref):
    a = p_ref[0] + p_ref[1] + g_ref[...]
    y_ref[...] = jnp.maximum(a * dinv_ref[...] + b_ref[...], 0.0)


def _k3(p, g, dinv, b):
    return pl.pallas_call(
        _k3_body,
        grid=(pl.cdiv(N, BR),),
        in_specs=[
            pl.BlockSpec((2, BR, D), lambda i: (0, i, 0)),
            pl.BlockSpec((BR, D), lambda i: (i, 0)),
            pl.BlockSpec((BR, 1), lambda i: (i, 0)),
            pl.BlockSpec((1, D), lambda i: (0, 0)),
        ],
        out_specs=pl.BlockSpec((BR, D), lambda i: (i, 0)),
        out_shape=jax.ShapeDtypeStruct((N, D), jnp.float32),
    )(p, g, dinv, b)


def kernel(x, edge_index, W1, b1, W2, b2):
    src_r = edge_index[0].reshape(ROWS_E, C)
    dst_r = edge_index[1].reshape(ROWS_E, C)
    padi = (N + jnp.arange(PAD_ROWS * C, dtype=jnp.int32)
            % (NPAD - N)).reshape(PAD_ROWS, C)
    zeros_nd = jnp.zeros((NPAD, D), jnp.float32)
    zeros_n = jnp.zeros((NPAD,), jnp.float32)
    ones_c = jnp.ones((CD,), jnp.float32)

    degp = _deg_kernel(dst_r, padi, ones_c, zeros_n)
    h1 = _k0(x, W1)
    degT = degp.T
    b1r = b1.reshape(1, D)
    b2r = b2.reshape(1, D)

    dinv, g1 = _k1(degT, h1)
    p1 = _edge_kernel(g1, src_r, dst_r, padi, zeros_nd)
    g2 = _k2(p1, g1, dinv, b1r, W2)
    p2 = _edge_kernel(g2, src_r, dst_r, padi, zeros_nd)
    return _k3(p2, g2, dinv, b2r)


# speedup vs baseline: 3.6475x; 1.0041x over previous
import functools

import jax
import jax.numpy as jnp
from jax import lax
from jax.experimental import pallas as pl
from jax.experimental.pallas import tpu as pltpu
from jax.experimental.pallas import tpu_sc as plsc

N = 10000
E = 320000
D = 128

NPAD = 10240
NW = 32
STRIPE = NPAD // 16

C = 128
CH = 80
EPAD = NW * CH * C

CD = 128
CHD = EPAD // (NW * CD)

ROWS_E = E // C
TILE_CH = CH
PAD_ROWS = NW * TILE_CH - ROWS_E
LAST_REAL = ROWS_E - (NW - 1) * TILE_CH

_mesh = plsc.VectorSubcoreMesh(core_axis_name="c", subcore_axis_name="s")


def _load_chunk_rows(wid, real_hbm, pad_hbm, dst_vmem, sem):
    def fire(j, carry):
        r = wid * TILE_CH + j

        @pl.when(r < ROWS_E)
        def _():
            pltpu.async_copy(real_hbm.at[r], dst_vmem.at[j], sem)

        @pl.when(r >= ROWS_E)
        def _():
            pltpu.async_copy(pad_hbm.at[r - ROWS_E], dst_vmem.at[j], sem)

        return carry

    def drain(j, carry):
        pltpu.make_async_copy(pad_hbm.at[0], dst_vmem.at[0], sem).wait()
        return carry

    lax.fori_loop(0, TILE_CH, fire, 0)
    lax.fori_loop(0, TILE_CH, drain, 0)


@functools.partial(
    pl.kernel,
    out_type=jax.ShapeDtypeStruct((2, NPAD), jnp.float32),
    mesh=_mesh,
    scratch_types=[
        pltpu.VMEM((CHD, CD), jnp.int32),
        pltpu.VMEM((CD,), jnp.float32),
        pltpu.VMEM_SHARED((NPAD,), jnp.float32),
        pltpu.SemaphoreType.DMA,
    ],
)
def _deg_kernel(dst_hbm, pad_hbm, ones_hbm, zn_hbm, out_hbm,
                dst_all, ones_v, deg_sh, sem):
    c = lax.axis_index("c")
    s = lax.axis_index("s")
    wid = s * 2 + c
    pltpu.sync_copy(zn_hbm.at[pl.ds(s * STRIPE, STRIPE)],
                    deg_sh.at[pl.ds(s * STRIPE, STRIPE)])
    _load_chunk_rows(wid, dst_hbm, pad_hbm, dst_all, sem)
    pltpu.sync_copy(ones_hbm, ones_v)
    plsc.subcore_barrier()

    def fire(j, carry):
        pltpu.async_copy(ones_v, deg_sh.at[dst_all.at[j]], sem, add=True)
        return carry

    def drain(j, carry):
        pltpu.make_async_copy(ones_v, deg_sh.at[pl.ds(0, CD)], sem).wait()
        return carry

    lax.fori_loop(0, CHD, fire, 0)
    lax.fori_loop(0, CHD, drain, 0)
    plsc.subcore_barrier()
    pltpu.sync_copy(deg_sh.at[pl.ds(s * STRIPE, STRIPE)],
                    out_hbm.at[c, pl.ds(s * STRIPE, STRIPE)])


@functools.partial(
    pl.kernel,
    out_type=jax.ShapeDtypeStruct((2, NPAD, D), jnp.float32),
    mesh=_mesh,
    scratch_types=[
        pltpu.VMEM((2, C), jnp.int32),
        pltpu.VMEM((CH, C), jnp.int32),
        pltpu.VMEM((2, C, D), jnp.float32),
        pltpu.VMEM_SHARED((NPAD, D), jnp.float32),
        pltpu.SemaphoreType.DMA,
        pltpu.SemaphoreType.DMA,
        pltpu.SemaphoreType.DMA,
        pltpu.SemaphoreType.DMA,
        pltpu.SemaphoreType.DMA,
    ],
)
def _edge_kernel(g_hbm, src_hbm, dst_hbm, pad_hbm, znd_hbm, out_hbm,
                 src_i, dst_all, rows, accum_sh, gsem, ssem0, ssem1,
                 isem0, isem1):
    c = lax.axis_index("c")
    s = lax.axis_index("s")
    wid = s * 2 + c
    pltpu.sync_copy(znd_hbm, rows.at[0])
    for q in range(STRIPE // C):
        pltpu.sync_copy(rows.at[0],
                        accum_sh.at[pl.ds(s * STRIPE + q * C, C)])
    _load_chunk_rows(wid, dst_hbm, pad_hbm, dst_all, gsem)
    plsc.subcore_barrier()

    isems = (isem0, isem1)
    ssems = (ssem0, ssem1)

    def i_start(j, b):
        r = wid * TILE_CH + j

        @pl.when(r < ROWS_E)
        def _():
            pltpu.async_copy(src_hbm.at[r], src_i.at[b], isems[b])

        @pl.when(r >= ROWS_E)
        def _():
            pltpu.async_copy(pad_hbm.at[r - ROWS_E], src_i.at[b], isems[b])

    def i_wait(b):
        pltpu.make_async_copy(pad_hbm.at[0], src_i.at[b], isems[b]).wait()

    def g_start(b):
        pltpu.async_copy(g_hbm.at[src_i.at[b]], rows.at[b], gsem)

    def g_wait(b):
        pltpu.make_async_copy(g_hbm.at[pl.ds(0, C)], rows.at[b], gsem).wait()

    def s_start(j, b):
        pltpu.async_copy(rows.at[b], accum_sh.at[dst_all.at[j]],
                         ssems[b], add=True)

    def s_wait(b):
        pltpu.make_async_copy(rows.at[b], accum_sh.at[pl.ds(0, C)],
                              ssems[b]).wait()

    i_start(0, 0)
    i_wait(0)
    g_start(0)
    i_start(1, 1)
    g_wait(0)
    i_start(2, 0)
    i_wait(1)
    g_start(1)
    s_start(0, 0)
    g_wait(1)
    i_start(3, 1)
    s_start(1, 1)
    s_wait(0)
    i_wait(0)
    g_start(0)

    def steady(t, carry):
        j = 2 * t
        g_wait(0)
        i_start(j + 2, 0)
        s_wait(1)
        i_wait(1)
        g_start(1)
        s_start(j, 0)
        g_wait(1)
        i_start(j + 3, 1)
        s_start(j + 1, 1)
        s_wait(0)
        i_wait(0)
        g_start(0)
        return carry

    lax.fori_loop(1, CH // 2 - 1, steady, 0)

    g_wait(0)
    s_wait(1)
    i_wait(1)
    g_start(1)
    s_start(CH - 2, 0)
    g_wait(1)
    s_start(CH - 1, 1)
    s_wait(0)
    s_wait(1)

    plsc.subcore_barrier()
    pltpu.sync_copy(accum_sh.at[pl.ds(s * STRIPE, STRIPE)],
                    out_hbm.at[c, pl.ds(s * STRIPE, STRIPE)])


BR = 1024


def _k0_body(x_ref, w_ref, h_ref):
    h_ref[...] = jnp.dot(x_ref[...], w_ref[...],
                         preferred_element_type=jnp.float32)


def _k0(x, w):
    return pl.pallas_call(
        _k0_body,
        grid=(NPAD // BR,),
        in_specs=[
            pl.BlockSpec((BR, D), lambda i: (i, 0)),
            pl.BlockSpec((D, D), lambda i: (0, 0)),
        ],
        out_specs=pl.BlockSpec((BR, D), lambda i: (i, 0)),
        out_shape=jax.ShapeDtypeStruct((NPAD, D), jnp.float32),
    )(x, w)


def _k1_body(degT_ref, h_ref, dinv_ref, g_ref):
    i = pl.program_id(0)
    rows = lax.broadcasted_iota(jnp.int32, (BR, 1), 0) + i * BR
    deg = degT_ref[:, 0:1] + degT_ref[:, 1:2] + 1.0
    dinv = lax.rsqrt(deg)
    dinv_ref[...] = dinv
    g_ref[...] = jnp.where(rows < N, h_ref[...] * dinv, 0.0)


def _k1(degT, h):
    return pl.pallas_call(
        _k1_body,
        grid=(NPAD // BR,),
        in_specs=[
            pl.BlockSpec((BR, 2), lambda i: (i, 0)),
            pl.BlockSpec((BR, D), lambda i: (i, 0)),
        ],
        out_specs=[
            pl.BlockSpec((BR, 1), lambda i: (i, 0)),
            pl.BlockSpec((BR, D), lambda i: (i, 0)),
        ],
        out_shape=[
            jax.ShapeDtypeStruct((NPAD, 1), jnp.float32),
            jax.ShapeDtypeStruct((NPAD, D), jnp.float32),
        ],
    )(degT, h)


def _k2_body(p_ref, g_ref, dinv_ref, b_ref, w_ref, gout_ref):
    i = pl.program_id(0)
    rows = lax.broadcasted_iota(jnp.int32, (BR, 1), 0) + i * BR
    a = p_ref[0] + p_ref[1] + g_ref[...]
    y = jnp.maximum(a * dinv_ref[...] + b_ref[...], 0.0)
    gout_ref[...] = jnp.where(
        rows < N,
        jnp.dot(y, w_ref[...],
                preferred_element_type=jnp.float32) * dinv_ref[...],
        0.0)


def _k2(p, g, dinv, b, w):
    return pl.pallas_call(
        _k2_body,
        grid=(NPAD // BR,),
        in_specs=[
            pl.BlockSpec((2, BR, D), lambda i: (0, i, 0)),
            pl.BlockSpec((BR, D), lambda i: (i, 0)),
            pl.BlockSpec((BR, 1), lambda i: (i, 0)),
            pl.BlockSpec((1, D), lambda i: (0, 0)),
            pl.BlockSpec((D, D), lambda i: (0, 0)),
        ],
        out_specs=pl.BlockSpec((BR, D), lambda i: (i, 0)),
        out_shape=jax.ShapeDtypeStruct((NPAD, D), jnp.float32),
    )(p, g, dinv, b, w)


def _k3_body(p_ref, g_ref, dinv_ref, b_ref, y_ref):
    a = p_ref[0] + p_ref[1] + g_ref[...]
    y_ref[...] = jnp.maximum(a * dinv_ref[...] + b_ref[...], 0.0)


def _k3(p, g, dinv, b):
    return pl.pallas_call(
        _k3_body,
        grid=(pl.cdiv(N, BR),),
        in_specs=[
            pl.BlockSpec((2, BR, D), lambda i: (0, i, 0)),
            pl.BlockSpec((BR, D), lambda i: (i, 0)),
            pl.BlockSpec((BR, 1), lambda i: (i, 0)),
            pl.BlockSpec((1, D), lambda i: (0, 0)),
        ],
        out_specs=pl.BlockSpec((BR, D), lambda i: (i, 0)),
        out_shape=jax.ShapeDtypeStruct((N, D), jnp.float32),
    )(p, g, dinv, b)


def kernel(x, edge_index, W1, b1, W2, b2):
    src_r = edge_index[0].reshape(ROWS_E, C)
    dst_r = edge_index[1].reshape(ROWS_E, C)
    padi = (N + jnp.arange(PAD_ROWS * C, dtype=jnp.int32)
            % (NPAD - N)).reshape(PAD_ROWS, C)
    zeros_nd = jnp.zeros((C, D), jnp.float32)
    zeros_n = jnp.zeros((NPAD,), jnp.float32)
    ones_c = jnp.ones((CD,), jnp.float32)

    degp = _deg_kernel(dst_r, padi, ones_c, zeros_n)
    h1 = _k0(x, W1)
    degT = degp.T
    b1r = b1.reshape(1, D)
    b2r = b2.reshape(1, D)

    dinv, g1 = _k1(degT, h1)
    p1 = _edge_kernel(g1, src_r, dst_r, padi, zeros_nd)
    g2 = _k2(p1, g1, dinv, b1r, W2)
    p2 = _edge_kernel(g2, src_r, dst_r, padi, zeros_nd)
    return _k3(p2, g2, dinv, b2r)


# speedup vs baseline: 3.7234x; 1.0208x over previous
import functools

import jax
import jax.numpy as jnp
from jax import lax
from jax.experimental import pallas as pl
from jax.experimental.pallas import tpu as pltpu
from jax.experimental.pallas import tpu_sc as plsc

N = 10000
E = 320000
D = 128

NPAD = 10240
NW = 32
STRIPE = NPAD // 16

C = 128
CH = 80
EPAD = NW * CH * C

CD = 128
CHD = EPAD // (NW * CD)

ROWS_E = E // C
TILE_CH = CH
PAD_ROWS = NW * TILE_CH - ROWS_E
LAST_REAL = ROWS_E - (NW - 1) * TILE_CH

_mesh = plsc.VectorSubcoreMesh(core_axis_name="c", subcore_axis_name="s")


def _load_chunk_rows(wid, real_hbm, pad_hbm, dst_vmem, sem):
    def fire(j, carry):
        r = wid * TILE_CH + j

        @pl.when(r < ROWS_E)
        def _():
            pltpu.async_copy(real_hbm.at[r], dst_vmem.at[j], sem)

        @pl.when(r >= ROWS_E)
        def _():
            pltpu.async_copy(pad_hbm.at[r - ROWS_E], dst_vmem.at[j], sem)

        return carry

    def drain(j, carry):
        pltpu.make_async_copy(pad_hbm.at[0], dst_vmem.at[0], sem).wait()
        return carry

    lax.fori_loop(0, TILE_CH, fire, 0)
    lax.fori_loop(0, TILE_CH, drain, 0)


@functools.partial(
    pl.kernel,
    out_type=jax.ShapeDtypeStruct((2, NPAD), jnp.float32),
    mesh=_mesh,
    scratch_types=[
        pltpu.VMEM((CHD, CD), jnp.int32),
        pltpu.VMEM((CD,), jnp.float32),
        pltpu.VMEM_SHARED((NPAD,), jnp.float32),
        pltpu.SemaphoreType.DMA,
    ],
)
def _deg_kernel(dst_hbm, pad_hbm, ones_hbm, zn_hbm, out_hbm,
                dst_all, ones_v, deg_sh, sem):
    c = lax.axis_index("c")
    s = lax.axis_index("s")
    wid = s * 2 + c
    pltpu.sync_copy(zn_hbm.at[pl.ds(s * STRIPE, STRIPE)],
                    deg_sh.at[pl.ds(s * STRIPE, STRIPE)])
    _load_chunk_rows(wid, dst_hbm, pad_hbm, dst_all, sem)
    pltpu.sync_copy(ones_hbm, ones_v)
    plsc.subcore_barrier()

    def fire(j, carry):
        pltpu.async_copy(ones_v, deg_sh.at[dst_all.at[j]], sem, add=True)
        return carry

    def drain(j, carry):
        pltpu.make_async_copy(ones_v, deg_sh.at[pl.ds(0, CD)], sem).wait()
        return carry

    lax.fori_loop(0, CHD, fire, 0)
    lax.fori_loop(0, CHD, drain, 0)
    plsc.subcore_barrier()
    pltpu.sync_copy(deg_sh.at[pl.ds(s * STRIPE, STRIPE)],
                    out_hbm.at[c, pl.ds(s * STRIPE, STRIPE)])


@functools.partial(
    pl.kernel,
    out_type=jax.ShapeDtypeStruct((2, NPAD, D), jnp.float32),
    mesh=_mesh,
    scratch_types=[
        pltpu.VMEM((2, C), jnp.int32),
        pltpu.VMEM((CH, C), jnp.int32),
        pltpu.VMEM((2, C, D), jnp.float32),
        pltpu.VMEM_SHARED((NPAD, D), jnp.float32),
        pltpu.SemaphoreType.DMA,
        pltpu.SemaphoreType.DMA,
        pltpu.SemaphoreType.DMA,
        pltpu.SemaphoreType.DMA,
        pltpu.SemaphoreType.DMA,
    ],
)
def _edge_kernel(g_hbm, src_hbm, dst_hbm, pad_hbm, znd_hbm, out_hbm,
                 src_i, dst_all, rows, accum_sh, gsem, ssem0, ssem1,
                 isem0, isem1):
    c = lax.axis_index("c")
    s = lax.axis_index("s")
    wid = s * 2 + c
    pltpu.sync_copy(znd_hbm, rows.at[0])
    for q in range(STRIPE // C):
        pltpu.sync_copy(rows.at[0],
                        accum_sh.at[pl.ds(s * STRIPE + q * C, C)])
    _load_chunk_rows(wid, dst_hbm, pad_hbm, dst_all, gsem)
    plsc.subcore_barrier()

    isems = (isem0, isem1)
    ssems = (ssem0, ssem1)

    def i_start(j, b):
        r = wid * TILE_CH + j

        @pl.when(r < ROWS_E)
        def _():
            pltpu.async_copy(src_hbm.at[r], src_i.at[b], isems[b])

        @pl.when(r >= ROWS_E)
        def _():
            pltpu.async_copy(pad_hbm.at[r - ROWS_E], src_i.at[b], isems[b])

    def i_wait(b):
        pltpu.make_async_copy(pad_hbm.at[0], src_i.at[b], isems[b]).wait()

    def g_start(b):
        pltpu.async_copy(g_hbm.at[src_i.at[b]], rows.at[b], gsem)

    def g_wait(b):
        pltpu.make_async_copy(g_hbm.at[pl.ds(0, C)], rows.at[b], gsem).wait()

    def s_start(j, b):
        pltpu.async_copy(rows.at[b], accum_sh.at[dst_all.at[j]],
                         ssems[b], add=True)

    def s_wait(b):
        pltpu.make_async_copy(rows.at[b], accum_sh.at[pl.ds(0, C)],
                              ssems[b]).wait()

    i_start(0, 0)
    i_wait(0)
    g_start(0)
    i_start(1, 1)
    g_wait(0)
    i_start(2, 0)
    i_wait(1)
    g_start(1)
    s_start(0, 0)
    g_wait(1)
    i_start(3, 1)
    s_start(1, 1)
    s_wait(0)
    i_wait(0)
    g_start(0)

    def steady(t, carry):
        j = 2 * t
        g_wait(0)
        i_start(j + 2, 0)
        s_wait(1)
        i_wait(1)
        g_start(1)
        s_start(j, 0)
        g_wait(1)
        i_start(j + 3, 1)
        s_start(j + 1, 1)
        s_wait(0)
        i_wait(0)
        g_start(0)
        return carry

    lax.fori_loop(1, CH // 2 - 1, steady, 0)

    g_wait(0)
    s_wait(1)
    i_wait(1)
    g_start(1)
    s_start(CH - 2, 0)
    g_wait(1)
    s_start(CH - 1, 1)
    s_wait(0)
    s_wait(1)

    plsc.subcore_barrier()
    pltpu.sync_copy(accum_sh.at[pl.ds(s * STRIPE, STRIPE)],
                    out_hbm.at[c, pl.ds(s * STRIPE, STRIPE)])


BR = 2048


def _k0_body(x_ref, w_ref, h_ref):
    h_ref[...] = jnp.dot(x_ref[...], w_ref[...],
                         preferred_element_type=jnp.float32)


def _k0(x, w):
    return pl.pallas_call(
        _k0_body,
        grid=(NPAD // BR,),
        in_specs=[
            pl.BlockSpec((BR, D), lambda i: (i, 0)),
            pl.BlockSpec((D, D), lambda i: (0, 0)),
        ],
        out_specs=pl.BlockSpec((BR, D), lambda i: (i, 0)),
        out_shape=jax.ShapeDtypeStruct((NPAD, D), jnp.float32),
    )(x, w)


def _k1_body(degT_ref, h_ref, dinv_ref, g_ref):
    i = pl.program_id(0)
    rows = lax.broadcasted_iota(jnp.int32, (BR, 1), 0) + i * BR
    deg = degT_ref[:, 0:1] + degT_ref[:, 1:2] + 1.0
    dinv = lax.rsqrt(deg)
    dinv_ref[...] = dinv
    g_ref[...] = jnp.where(rows < N, h_ref[...] * dinv, 0.0)


def _k1(degT, h):
    return pl.pallas_call(
        _k1_body,
        grid=(NPAD // BR,),
        in_specs=[
            pl.BlockSpec((BR, 2), lambda i: (i, 0)),
            pl.BlockSpec((BR, D), lambda i: (i, 0)),
        ],
        out_specs=[
            pl.BlockSpec((BR, 1), lambda i: (i, 0)),
            pl.BlockSpec((BR, D), lambda i: (i, 0)),
        ],
        out_shape=[
            jax.ShapeDtypeStruct((NPAD, 1), jnp.float32),
            jax.ShapeDtypeStruct((NPAD, D), jnp.float32),
        ],
    )(degT, h)


def _k2_body(p_ref, g_ref, dinv_ref, b_ref, w_ref, gout_ref):
    i = pl.program_id(0)
    rows = lax.broadcasted_iota(jnp.int32, (BR, 1), 0) + i * BR
    a = p_ref[0] + p_ref[1] + g_ref[...]
    y = jnp.maximum(a * dinv_ref[...] + b_ref[...], 0.0)
    gout_ref[...] = jnp.where(
        rows < N,
        jnp.dot(y, w_ref[...],
                preferred_element_type=jnp.float32) * dinv_ref[...],
        0.0)


def _k2(p, g, dinv, b, w):
    return pl.pallas_call(
        _k2_body,
        grid=(NPAD // BR,),
        in_specs=[
            pl.BlockSpec((2, BR, D), lambda i: (0, i, 0)),
            pl.BlockSpec((BR, D), lambda i: (i, 0)),
            pl.BlockSpec((BR, 1), lambda i: (i, 0)),
            pl.BlockSpec((1, D), lambda i: (0, 0)),
            pl.BlockSpec((D, D), lambda i: (0, 0)),
        ],
        out_specs=pl.BlockSpec((BR, D), lambda i: (i, 0)),
        out_shape=jax.ShapeDtypeStruct((NPAD, D), jnp.float32),
    )(p, g, dinv, b, w)


def _k3_body(p_ref, g_ref, dinv_ref, b_ref, y_ref):
    a = p_ref[0] + p_ref[1] + g_ref[...]
    y_ref[...] = jnp.maximum(a * dinv_ref[...] + b_ref[...], 0.0)


def _k3(p, g, dinv, b):
    return pl.pallas_call(
        _k3_body,
        grid=(pl.cdiv(N, BR),),
        in_specs=[
            pl.BlockSpec((2, BR, D), lambda i: (0, i, 0)),
            pl.BlockSpec((BR, D), lambda i: (i, 0)),
            pl.BlockSpec((BR, 1), lambda i: (i, 0)),
            pl.BlockSpec((1, D), lambda i: (0, 0)),
        ],
        out_specs=pl.BlockSpec((BR, D), lambda i: (i, 0)),
        out_shape=jax.ShapeDtypeStruct((N, D), jnp.float32),
    )(p, g, dinv, b)


def kernel(x, edge_index, W1, b1, W2, b2):
    src_r = edge_index[0].reshape(ROWS_E, C)
    dst_r = edge_index[1].reshape(ROWS_E, C)
    padi = (N + jnp.arange(PAD_ROWS * C, dtype=jnp.int32)
            % (NPAD - N)).reshape(PAD_ROWS, C)
    zeros_nd = jnp.zeros((C, D), jnp.float32)
    zeros_n = jnp.zeros((NPAD,), jnp.float32)
    ones_c = jnp.ones((CD,), jnp.float32)

    degp = _deg_kernel(dst_r, padi, ones_c, zeros_n)
    h1 = _k0(x, W1)
    degT = degp.T
    b1r = b1.reshape(1, D)
    b2r = b2.reshape(1, D)

    dinv, g1 = _k1(degT, h1)
    p1 = _edge_kernel(g1, src_r, dst_r, padi, zeros_nd)
    g2 = _k2(p1, g1, dinv, b1r, W2)
    p2 = _edge_kernel(g2, src_r, dst_r, padi, zeros_nd)
    return _k3(p2, g2, dinv, b2r)
